# baseline (speedup 1.0000x reference)
import functools

import jax
import jax.numpy as jnp
from jax import lax
from jax.experimental import pallas as pl
from jax.experimental.pallas import tpu as pltpu
from jax.experimental.pallas import tpu_sc as plsc

_N = 4
_LOW, _HIGH = 0.0, 20.0
_B, _T, _C, _D, _L, _K = 32, 128, 4, 4096, 64, 5
_R = _C * _L
_NW = 32
_W = _D // _NW
_H = 16
_WH = _W + _H


def _prep_body(x_ref, lvl_ref, ch_ref, fidx_ref, bound_ref):
    xq = x_ref[:].reshape(_B * _T, _C)
    idx = jnp.clip(
        jnp.round((xq - _LOW) / (_HIGH - _LOW) * (_L - 1)), 0, _L - 1
    ).astype(jnp.int32)
    fidx_ref[:] = idx + lax.broadcasted_iota(jnp.int32, (_B * _T, _C), 1) * _L
    lvl = lvl_ref[:]
    ch = ch_ref[:]
    bound_ref[:] = (ch[:, None, :] * lvl[None, :, :]).reshape(_R, _D)


def _splat16(v, k):
    idx = jnp.full((16,), k, jnp.int32)
    return lax.gather(
        v,
        idx[:, None],
        lax.GatherDimensionNumbers(
            offset_dims=(), collapsed_slice_dims=(0,), start_index_map=(0,)
        ),
        (1,),
        mode=lax.GatherScatterMode.PROMISE_IN_BOUNDS,
    )


def _sc_body(tabs_hbm, fidx_hbm, out_hbm, tab_v, fidx_v, s_v, acc_v):
    w = lax.axis_index("s") * 2 + lax.axis_index("c")
    pltpu.sync_copy(tabs_hbm.at[w], tab_v)
    pltpu.sync_copy(fidx_hbm, fidx_v)
    zero = jnp.zeros((16,), jnp.float32)

    def per_b(b, _):
        def per_row(g2, _):
            for gg in range(8):
                vec = fidx_v[b * 4 + g2, pl.ds(gg * 16, 16)]
                for tq in range(4):
                    t = g2 * 32 + gg * 4 + tq
                    rows = [_splat16(vec, tq * 4 + c) for c in range(_C)]
                    for j in range(_WH // 16):
                        cols = jnp.full((16,), j * 16, jnp.int32) + lax.iota(
                            jnp.int32, 16
                        )
                        v0, v1, v2, v3 = (
                            plsc.load_gather(tab_v, [rows[c], cols])
                            for c in range(_C)
                        )
                        s_v[t, pl.ds(j * 16, 16)] = (v0 + v1) + (v2 + v3)
            return 0

        lax.fori_loop(0, 4, per_row, 0, unroll=False)

        def per_win(t, accs):
            new_accs = []
            for j in range(_W // 16):
                o = _H + j * 16
                p = (s_v[t, pl.ds(o - 3, 16)] * s_v[t + 1, pl.ds(o - 2, 16)]) * (
                    s_v[t + 2, pl.ds(o - 1, 16)] * s_v[t + 3, pl.ds(o, 16)]
                )
                new_accs.append(accs[j] + p)
            return tuple(new_accs)

        accs = lax.fori_loop(
            0, _T - (_N - 1), per_win, (zero,) * (_W // 16), unroll=False
        )
        for j in range(_W // 16):
            acc_v[b, pl.ds(j * 16, 16)] = accs[j]
        return 0

    lax.fori_loop(0, _B, per_b, 0, unroll=False)
    pltpu.sync_copy(acc_v, out_hbm.at[w])


def _finish_body(enc_ref, am_ref, out_ref):
    enc = enc_ref[:]
    am = am_ref[:]
    enc_norm = jnp.sqrt(jnp.sum(enc * enc, axis=1, keepdims=True)) + 1e-12
    am_norm = jnp.sqrt(jnp.sum(am * am, axis=1, keepdims=True)) + 1e-12
    dots = lax.dot_general(
        enc, am, (((1,), (1,)), ((), ())), preferred_element_type=jnp.float32
    )
    out_ref[:] = dots / (enc_norm * am_norm.T)


def kernel(x, level_table, channel_weight, am_weight):
    fidx, bound = pl.pallas_call(
        _prep_body,
        out_shape=(
            jax.ShapeDtypeStruct((_B * _T, _C), jnp.int32),
            jax.ShapeDtypeStruct((_R, _D), jnp.float32),
        ),
    )(x, level_table, channel_weight)

    rolled = jnp.concatenate([bound[:, _D - _H :], bound[:, : _D - _H]], axis=1)
    main = bound.reshape(_R, _NW, _W).transpose(1, 0, 2)
    halo = rolled.reshape(_R, _NW, _W)[:, :, :_H].transpose(1, 0, 2)
    tabs = jnp.concatenate([halo, main], axis=2)

    sc = functools.partial(
        pl.kernel,
        out_type=jax.ShapeDtypeStruct((_NW, _B, _W), jnp.float32),
        mesh=plsc.VectorSubcoreMesh(core_axis_name="c", subcore_axis_name="s"),
        scratch_types=[
            pltpu.VMEM((_R, _WH), jnp.float32),
            pltpu.VMEM((_B * _T * _C // 128, 128), jnp.int32),
            pltpu.VMEM((_T, _WH), jnp.float32),
            pltpu.VMEM((_B, _W), jnp.float32),
        ],
        compiler_params=pltpu.CompilerParams(
            use_tc_tiling_on_sc=False, needs_layout_passes=False
        ),
    )(_sc_body)
    enc_chunks = sc(tabs, fidx.reshape(_B * _T * _C // 128, 128))
    enc = enc_chunks.transpose(1, 0, 2).reshape(_B, _D)

    return pl.pallas_call(
        _finish_body,
        out_shape=jax.ShapeDtypeStruct((_B, _K), jnp.float32),
    )(enc, am_weight)


# speedup vs baseline: 1.0858x; 1.0858x over previous
import functools

import jax
import jax.numpy as jnp
from jax import lax
from jax.experimental import pallas as pl
from jax.experimental.pallas import tpu as pltpu
from jax.experimental.pallas import tpu_sc as plsc

_N = 4
_LOW, _HIGH = 0.0, 20.0
_B, _T, _C, _D, _L, _K = 32, 128, 4, 4096, 64, 5
_R = _C * _L
_NW = 32
_W = _D // _NW
_H = 16
_WH = _W + _H


def _prep_body(x_ref, lvl_ref, ch_ref, fidx_ref, bound_ref):
    xq = x_ref[:].reshape(_B * _T, _C)
    idx = jnp.clip(
        jnp.round((xq - _LOW) / (_HIGH - _LOW) * (_L - 1)), 0, _L - 1
    ).astype(jnp.int32)
    fidx_ref[:] = idx + lax.broadcasted_iota(jnp.int32, (_B * _T, _C), 1) * _L
    lvl = lvl_ref[:]
    ch = ch_ref[:]
    bound_ref[:] = (ch[:, None, :] * lvl[None, :, :]).reshape(_R, _D)


def _splat16(v, k):
    idx = jnp.full((16,), k, jnp.int32)
    return lax.gather(
        v,
        idx[:, None],
        lax.GatherDimensionNumbers(
            offset_dims=(), collapsed_slice_dims=(0,), start_index_map=(0,)
        ),
        (1,),
        mode=lax.GatherScatterMode.PROMISE_IN_BOUNDS,
    )


def _sc_body(tabs_hbm, fidx_hbm, out_hbm, tab_v, fidx_v, s_v, acc_v):
    w = lax.axis_index("s") * 2 + lax.axis_index("c")
    pltpu.sync_copy(tabs_hbm.at[w], tab_v)
    pltpu.sync_copy(fidx_hbm, fidx_v)
    zero = jnp.zeros((16,), jnp.float32)

    def per_b(b, _):
        def per_row(g2, _):
            for gg in range(8):
                vec = fidx_v[b * 4 + g2, pl.ds(gg * 16, 16)]
                for tq in range(4):
                    t = g2 * 32 + gg * 4 + tq
                    i0 = vec[tq * 4 + 0]
                    i1 = vec[tq * 4 + 1]
                    i2 = vec[tq * 4 + 2]
                    i3 = vec[tq * 4 + 3]
                    for j in range(_WH // 16):
                        sl = pl.ds(j * 16, 16)
                        s_v[t, sl] = (tab_v[i0, sl] + tab_v[i1, sl]) + (
                            tab_v[i2, sl] + tab_v[i3, sl]
                        )
            return 0

        lax.fori_loop(0, 4, per_row, 0, unroll=2)

        def per_win(t, accs):
            new_accs = []
            for j in range(_W // 16):
                o = _H + j * 16
                p = (s_v[t, pl.ds(o - 3, 16)] * s_v[t + 1, pl.ds(o - 2, 16)]) * (
                    s_v[t + 2, pl.ds(o - 1, 16)] * s_v[t + 3, pl.ds(o, 16)]
                )
                new_accs.append(accs[j] + p)
            return tuple(new_accs)

        accs = lax.fori_loop(
            0, _T - (_N - 1), per_win, (zero,) * (_W // 16), unroll=5
        )
        for j in range(_W // 16):
            acc_v[b, pl.ds(j * 16, 16)] = accs[j]
        return 0

    lax.fori_loop(0, _B, per_b, 0, unroll=False)
    pltpu.sync_copy(acc_v, out_hbm.at[w])


def _finish_body(enc_ref, am_ref, out_ref):
    enc = enc_ref[:]
    am = am_ref[:]
    enc_norm = jnp.sqrt(jnp.sum(enc * enc, axis=1, keepdims=True)) + 1e-12
    am_norm = jnp.sqrt(jnp.sum(am * am, axis=1, keepdims=True)) + 1e-12
    dots = lax.dot_general(
        enc, am, (((1,), (1,)), ((), ())), preferred_element_type=jnp.float32
    )
    out_ref[:] = dots / (enc_norm * am_norm.T)


def kernel(x, level_table, channel_weight, am_weight):
    fidx, bound = pl.pallas_call(
        _prep_body,
        out_shape=(
            jax.ShapeDtypeStruct((_B * _T, _C), jnp.int32),
            jax.ShapeDtypeStruct((_R, _D), jnp.float32),
        ),
    )(x, level_table, channel_weight)

    rolled = jnp.concatenate([bound[:, _D - _H :], bound[:, : _D - _H]], axis=1)
    main = bound.reshape(_R, _NW, _W).transpose(1, 0, 2)
    halo = rolled.reshape(_R, _NW, _W)[:, :, :_H].transpose(1, 0, 2)
    tabs = jnp.concatenate([halo, main], axis=2)

    sc = functools.partial(
        pl.kernel,
        out_type=jax.ShapeDtypeStruct((_NW, _B, _W), jnp.float32),
        mesh=plsc.VectorSubcoreMesh(core_axis_name="c", subcore_axis_name="s"),
        scratch_types=[
            pltpu.VMEM((_R, _WH), jnp.float32),
            pltpu.VMEM((_B * _T * _C // 128, 128), jnp.int32),
            pltpu.VMEM((_T, _WH), jnp.float32),
            pltpu.VMEM((_B, _W), jnp.float32),
        ],
        compiler_params=pltpu.CompilerParams(
            use_tc_tiling_on_sc=False, needs_layout_passes=False
        ),
    )(_sc_body)
    enc_chunks = sc(tabs, fidx.reshape(_B * _T * _C // 128, 128))
    enc = enc_chunks.transpose(1, 0, 2).reshape(_B, _D)

    return pl.pallas_call(
        _finish_body,
        out_shape=jax.ShapeDtypeStruct((_B, _K), jnp.float32),
    )(enc, am_weight)


# speedup vs baseline: 1.1957x; 1.1012x over previous
import functools

import jax
import jax.numpy as jnp
from jax import lax
from jax.experimental import pallas as pl
from jax.experimental.pallas import tpu as pltpu
from jax.experimental.pallas import tpu_sc as plsc

_N = 4
_LOW, _HIGH = 0.0, 20.0
_B, _T, _C, _D, _L, _K = 32, 128, 4, 4096, 64, 5
_R = _C * _L
_NW = 32
_W = _D // _NW
_H = 16
_WH = _W + _H
_CH = 128
_NCH = _B * _T * _C // _CH


def _prep_body(x_ref, lvl_ref, ch_ref, fidx_ref, bound_ref):
    xq = x_ref[:].reshape(_B * _T, _C)
    idx = jnp.clip(
        jnp.round((xq - _LOW) / (_HIGH - _LOW) * (_L - 1)), 0, _L - 1
    ).astype(jnp.int32)
    fidx_ref[:] = idx + lax.broadcasted_iota(jnp.int32, (_B * _T, _C), 1) * _L
    lvl = lvl_ref[:]
    ch = ch_ref[:]
    bound_ref[:] = (ch[:, None, :] * lvl[None, :, :]).reshape(_R, _D)


def _sc_body(tabs_hbm, fidx_hbm, out_hbm, idx_v, buf0, buf1, s_v, acc_v, sem0, sem1):
    w = lax.axis_index("s") * 2 + lax.axis_index("c")
    pltpu.sync_copy(fidx_hbm, idx_v)
    woff = jnp.full((16,), w * _R, jnp.int32)

    def add_off(i, _):
        sl = pl.ds(i * 16, 16)
        idx_v[sl] = idx_v[sl] + woff
        return 0

    lax.fori_loop(0, _B * _T * _C // 16, add_off, 0, unroll=False)

    bufs = (buf0, buf1)
    sems = (sem0, sem1)

    def issue(c, k):
        pltpu.async_copy(tabs_hbm.at[idx_v.at[pl.ds(c * _CH, _CH)]], bufs[k], sems[k])

    def wait(k):
        pltpu.make_async_copy(tabs_hbm.at[pl.ds(0, _CH)], bufs[k], sems[k]).wait()

    issue(0, 0)
    zero = jnp.zeros((16,), jnp.float32)

    def per_b(b, _):
        for q in range(4):
            c = b * 4 + q
            k = q % 2
            wait(k)

            @pl.when(c + 1 < _NCH)
            def _():
                issue(c + 1, (q + 1) % 2)

            buf = bufs[k]
            for tt in range(_T // 4):
                t = q * (_T // 4) + tt
                for j in range(_WH // 16):
                    sl = pl.ds(j * 16, 16)
                    s_v[t, sl] = (buf[tt * 4 + 0, sl] + buf[tt * 4 + 1, sl]) + (
                        buf[tt * 4 + 2, sl] + buf[tt * 4 + 3, sl]
                    )

        def per_win(t, accs):
            new_accs = []
            for j in range(_W // 16):
                o = _H + j * 16
                p = (s_v[t, pl.ds(o - 3, 16)] * s_v[t + 1, pl.ds(o - 2, 16)]) * (
                    s_v[t + 2, pl.ds(o - 1, 16)] * s_v[t + 3, pl.ds(o, 16)]
                )
                new_accs.append(accs[j] + p)
            return tuple(new_accs)

        accs = lax.fori_loop(
            0, _T - (_N - 1), per_win, (zero,) * (_W // 16), unroll=False
        )
        for j in range(_W // 16):
            acc_v[b, pl.ds(j * 16, 16)] = accs[j]
        return 0

    lax.fori_loop(0, _B, per_b, 0, unroll=False)
    pltpu.sync_copy(acc_v, out_hbm.at[w])


def _finish_body(enc_ref, am_ref, out_ref):
    enc = enc_ref[:]
    am = am_ref[:]
    enc_norm = jnp.sqrt(jnp.sum(enc * enc, axis=1, keepdims=True)) + 1e-12
    am_norm = jnp.sqrt(jnp.sum(am * am, axis=1, keepdims=True)) + 1e-12
    dots = lax.dot_general(
        enc, am, (((1,), (1,)), ((), ())), preferred_element_type=jnp.float32
    )
    out_ref[:] = dots / (enc_norm * am_norm.T)


def kernel(x, level_table, channel_weight, am_weight):
    fidx, bound = pl.pallas_call(
        _prep_body,
        out_shape=(
            jax.ShapeDtypeStruct((_B * _T, _C), jnp.int32),
            jax.ShapeDtypeStruct((_R, _D), jnp.float32),
        ),
    )(x, level_table, channel_weight)

    rolled = jnp.concatenate([bound[:, _D - _H :], bound[:, : _D - _H]], axis=1)
    main = bound.reshape(_R, _NW, _W).transpose(1, 0, 2)
    halo = rolled.reshape(_R, _NW, _W)[:, :, :_H].transpose(1, 0, 2)
    tabs = jnp.concatenate([halo, main], axis=2)

    sc = functools.partial(
        pl.kernel,
        out_type=jax.ShapeDtypeStruct((_NW, _B, _W), jnp.float32),
        mesh=plsc.VectorSubcoreMesh(core_axis_name="c", subcore_axis_name="s"),
        scratch_types=[
            pltpu.VMEM((_B * _T * _C,), jnp.int32),
            pltpu.VMEM((_CH, _WH), jnp.float32),
            pltpu.VMEM((_CH, _WH), jnp.float32),
            pltpu.VMEM((_T, _WH), jnp.float32),
            pltpu.VMEM((_B, _W), jnp.float32),
            pltpu.SemaphoreType.DMA,
            pltpu.SemaphoreType.DMA,
        ],
        compiler_params=pltpu.CompilerParams(
            use_tc_tiling_on_sc=False, needs_layout_passes=False
        ),
    )(_sc_body)
    enc_chunks = sc(
        tabs.reshape(_NW * _R, _WH), fidx.reshape(_B * _T * _C)
    )
    enc = enc_chunks.transpose(1, 0, 2).reshape(_B, _D)

    return pl.pallas_call(
        _finish_body,
        out_shape=jax.ShapeDtypeStruct((_B, _K), jnp.float32),
    )(enc, am_weight)


# speedup vs baseline: 1.2402x; 1.0372x over previous
import functools

import jax
import jax.numpy as jnp
from jax import lax
from jax.experimental import pallas as pl
from jax.experimental.pallas import tpu as pltpu
from jax.experimental.pallas import tpu_sc as plsc

_N = 4
_LOW, _HIGH = 0.0, 20.0
_B, _T, _C, _D, _L, _K = 32, 128, 4, 4096, 64, 5
_R = _C * _L
_NW = 32
_W = _D // _NW
_H = 16
_WH = _W + _H


def _prep_body(x_ref, lvl_ref, ch_ref, fidx_ref, bound_ref):
    xq = x_ref[:].reshape(_B * _T, _C)
    idx = jnp.clip(
        jnp.round((xq - _LOW) / (_HIGH - _LOW) * (_L - 1)), 0, _L - 1
    ).astype(jnp.int32)
    fidx_ref[:] = idx + lax.broadcasted_iota(jnp.int32, (_B * _T, _C), 1) * _L
    lvl = lvl_ref[:]
    ch = ch_ref[:]
    bound_ref[:] = (ch[:, None, :] * lvl[None, :, :]).reshape(_R, _D)


def _sc_body(tabs_hbm, fidx_hbm, out_hbm, tab_v, fidx_v, s_v, acc_v):
    w = lax.axis_index("s") * 2 + lax.axis_index("c")
    pltpu.sync_copy(tabs_hbm.at[w], tab_v)
    pltpu.sync_copy(fidx_hbm, fidx_v)
    zero = jnp.zeros((16,), jnp.float32)

    def per_b(b, _):
        @plsc.parallel_loop(0, 4)
        def per_row(g2):
            for gg in range(8):
                vec = fidx_v[b * 4 + g2, pl.ds(gg * 16, 16)]
                for tq in range(4):
                    t = g2 * 32 + gg * 4 + tq
                    i0 = vec[tq * 4 + 0]
                    i1 = vec[tq * 4 + 1]
                    i2 = vec[tq * 4 + 2]
                    i3 = vec[tq * 4 + 3]
                    for j in range(_WH // 16):
                        sl = pl.ds(j * 16, 16)
                        s_v[t, sl] = (tab_v[i0, sl] + tab_v[i1, sl]) + (
                            tab_v[i2, sl] + tab_v[i3, sl]
                        )

        @plsc.parallel_loop(0, _T - (_N - 1), carry=(zero,) * (_W // 16))
        def accs(t, acc):
            new_accs = []
            for j in range(_W // 16):
                o = _H + j * 16
                p = (s_v[t, pl.ds(o - 3, 16)] * s_v[t + 1, pl.ds(o - 2, 16)]) * (
                    s_v[t + 2, pl.ds(o - 1, 16)] * s_v[t + 3, pl.ds(o, 16)]
                )
                new_accs.append(acc[j] + p)
            return tuple(new_accs)

        for j in range(_W // 16):
            acc_v[b, pl.ds(j * 16, 16)] = accs[j]
        return 0

    lax.fori_loop(0, _B, per_b, 0, unroll=False)
    pltpu.sync_copy(acc_v, out_hbm.at[w])


def _finish_body(enc_ref, am_ref, out_ref):
    enc = enc_ref[:]
    am = am_ref[:]
    enc_norm = jnp.sqrt(jnp.sum(enc * enc, axis=1, keepdims=True)) + 1e-12
    am_norm = jnp.sqrt(jnp.sum(am * am, axis=1, keepdims=True)) + 1e-12
    dots = lax.dot_general(
        enc, am, (((1,), (1,)), ((), ())), preferred_element_type=jnp.float32
    )
    out_ref[:] = dots / (enc_norm * am_norm.T)


def kernel(x, level_table, channel_weight, am_weight):
    fidx, bound = pl.pallas_call(
        _prep_body,
        out_shape=(
            jax.ShapeDtypeStruct((_B * _T, _C), jnp.int32),
            jax.ShapeDtypeStruct((_R, _D), jnp.float32),
        ),
    )(x, level_table, channel_weight)

    rolled = jnp.concatenate([bound[:, _D - _H :], bound[:, : _D - _H]], axis=1)
    main = bound.reshape(_R, _NW, _W).transpose(1, 0, 2)
    halo = rolled.reshape(_R, _NW, _W)[:, :, :_H].transpose(1, 0, 2)
    tabs = jnp.concatenate([halo, main], axis=2)

    sc = functools.partial(
        pl.kernel,
        out_type=jax.ShapeDtypeStruct((_NW, _B, _W), jnp.float32),
        mesh=plsc.VectorSubcoreMesh(core_axis_name="c", subcore_axis_name="s"),
        scratch_types=[
            pltpu.VMEM((_R, _WH), jnp.float32),
            pltpu.VMEM((_B * _T * _C // 128, 128), jnp.int32),
            pltpu.VMEM((_T, _WH), jnp.float32),
            pltpu.VMEM((_B, _W), jnp.float32),
        ],
        compiler_params=pltpu.CompilerParams(
            use_tc_tiling_on_sc=False, needs_layout_passes=False
        ),
    )(_sc_body)
    enc_chunks = sc(
        tabs, fidx.reshape(_B * _T * _C // 128, 128)
    )
    enc = enc_chunks.transpose(1, 0, 2).reshape(_B, _D)

    return pl.pallas_call(
        _finish_body,
        out_shape=jax.ShapeDtypeStruct((_B, _K), jnp.float32),
    )(enc, am_weight)


# speedup vs baseline: 3.2941x; 2.6561x over previous
import functools

import jax
import jax.numpy as jnp
from jax import lax
from jax.experimental import pallas as pl
from jax.experimental.pallas import tpu as pltpu
from jax.experimental.pallas import tpu_sc as plsc

_N = 4
_LOW, _HIGH = 0.0, 20.0
_B, _T, _C, _D, _L, _K = 32, 128, 4, 4096, 64, 5
_R = _C * _L
_NW = 32
_W = _D // _NW
_H = 16
_WH = _W + _H


def _prep_body(x_ref, lvl_ref, ch_ref, fidx_ref, bound_ref):
    xq = x_ref[:].reshape(_B * _T, _C)
    idx = jnp.clip(
        jnp.round((xq - _LOW) / (_HIGH - _LOW) * (_L - 1)), 0, _L - 1
    ).astype(jnp.int32)
    fidx_ref[:] = idx + lax.broadcasted_iota(jnp.int32, (_B * _T, _C), 1) * _L
    lvl = lvl_ref[:]
    ch = ch_ref[:]
    bound_ref[:] = (ch[:, None, :] * lvl[None, :, :]).reshape(_R, _D)


def _sc_body(tabs_hbm, fidx_hbm, out_hbm, tab_v, fidx_v, s_v, acc_v):
    w = lax.axis_index("s") * 2 + lax.axis_index("c")
    pltpu.sync_copy(tabs_hbm.at[w], tab_v)
    pltpu.sync_copy(fidx_hbm, fidx_v)
    zero = jnp.zeros((16,), jnp.float32)

    def per_b(b, _):
        @plsc.parallel_loop(0, _T, unroll=4)
        def per_t(t):
            vec = fidx_v[pl.ds((b * _T + t) * _C, 16)]
            i0 = vec[0]
            i1 = vec[1]
            i2 = vec[2]
            i3 = vec[3]
            for j in range(_WH // 16):
                sl = pl.ds(j * 16, 16)
                s_v[t, sl] = (tab_v[i0, sl] + tab_v[i1, sl]) + (
                    tab_v[i2, sl] + tab_v[i3, sl]
                )

        @plsc.parallel_loop(0, _T - (_N - 1), carry=(zero,) * (_W // 16))
        def accs(t, acc):
            new_accs = []
            for j in range(_W // 16):
                o = _H + j * 16
                p = (s_v[t, pl.ds(o - 3, 16)] * s_v[t + 1, pl.ds(o - 2, 16)]) * (
                    s_v[t + 2, pl.ds(o - 1, 16)] * s_v[t + 3, pl.ds(o, 16)]
                )
                new_accs.append(acc[j] + p)
            return tuple(new_accs)

        for j in range(_W // 16):
            acc_v[b, pl.ds(j * 16, 16)] = accs[j]
        return 0

    lax.fori_loop(0, _B, per_b, 0, unroll=False)
    pltpu.sync_copy(acc_v, out_hbm.at[w])


def _finish_body(enc_ref, am_ref, out_ref):
    enc = enc_ref[:]
    am = am_ref[:]
    enc_norm = jnp.sqrt(jnp.sum(enc * enc, axis=1, keepdims=True)) + 1e-12
    am_norm = jnp.sqrt(jnp.sum(am * am, axis=1, keepdims=True)) + 1e-12
    dots = lax.dot_general(
        enc, am, (((1,), (1,)), ((), ())), preferred_element_type=jnp.float32
    )
    out_ref[:] = dots / (enc_norm * am_norm.T)


def kernel(x, level_table, channel_weight, am_weight):
    fidx, bound = pl.pallas_call(
        _prep_body,
        out_shape=(
            jax.ShapeDtypeStruct((_B * _T, _C), jnp.int32),
            jax.ShapeDtypeStruct((_R, _D), jnp.float32),
        ),
    )(x, level_table, channel_weight)

    rolled = jnp.concatenate([bound[:, _D - _H :], bound[:, : _D - _H]], axis=1)
    main = bound.reshape(_R, _NW, _W).transpose(1, 0, 2)
    halo = rolled.reshape(_R, _NW, _W)[:, :, :_H].transpose(1, 0, 2)
    tabs = jnp.concatenate([halo, main], axis=2)

    sc = functools.partial(
        pl.kernel,
        out_type=jax.ShapeDtypeStruct((_NW, _B, _W), jnp.float32),
        mesh=plsc.VectorSubcoreMesh(core_axis_name="c", subcore_axis_name="s"),
        scratch_types=[
            pltpu.VMEM((_R, _WH), jnp.float32),
            pltpu.VMEM((_B * _T * _C,), jnp.int32),
            pltpu.VMEM((_T, _WH), jnp.float32),
            pltpu.VMEM((_B, _W), jnp.float32),
        ],
        compiler_params=pltpu.CompilerParams(
            use_tc_tiling_on_sc=False, needs_layout_passes=False
        ),
    )(_sc_body)
    enc_chunks = sc(tabs, fidx.reshape(_B * _T * _C))
    enc = enc_chunks.transpose(1, 0, 2).reshape(_B, _D)

    return pl.pallas_call(
        _finish_body,
        out_shape=jax.ShapeDtypeStruct((_B, _K), jnp.float32),
    )(enc, am_weight)
